# Initial kernel scaffold; baseline (speedup 1.0000x reference)
import jax
import jax.numpy as jnp
from jax.experimental import pallas as pl


def kernel(x, edge_index, batch, W1, a_src1, a_dst1, b1, W2, a_src2, a_dst2, b2, W3, a_src3, a_dst3, b3):
    raise NotImplementedError("write your pallas kernel here")


# speedup vs baseline: 13.0428x; 13.0428x over previous
import functools

import jax
import jax.numpy as jnp
from jax import lax
from jax.experimental import pallas as pl
from jax.experimental.pallas import tpu as pltpu
from jax.experimental.pallas import tpu_sc as plsc

N = 10000
E = 320000
D = 128
NG = 128
NEG = 0.2

NC, NS, L = 2, 16, 16
NW = NC * NS
NP = 10240
K = 128
NCHUNK = 81
EPT = NCHUNK * K
E2P = NW * EPT
SL = NP // NS


def _leaky(v):
    return jnp.maximum(v, NEG * v)


def _gmax_of(as_v):
    def body(i, acc):
        return jnp.maximum(acc, as_v[pl.ds(i * L, L)])
    m = lax.fori_loop(0, NP // L, body, jnp.full((L,), -jnp.inf, jnp.float32))
    return jnp.max(m)


def _phase_a_body(src_hbm, dst_hbm, as_hbm, ad_hbm, den_hbm, ex_hbm,
                  as_v, ad_v, srcv, dstv, exv, dloc, accv, tmpv, dsh, sem):
    cid = lax.axis_index("c")
    sid = lax.axis_index("s")
    wid = cid * NS + sid
    pltpu.sync_copy(as_hbm, as_v)
    pltpu.sync_copy(ad_hbm, ad_v)
    pltpu.sync_copy(src_hbm.at[pl.ds(wid * EPT, EPT)], srcv)
    pltpu.sync_copy(dst_hbm.at[pl.ds(wid * EPT, EPT)], dstv)

    def zero(i, _):
        dloc[pl.ds(i * L, L)] = jnp.zeros((L,), jnp.float32)
        return _
    lax.fori_loop(0, NP // L, zero, None)

    gmax = _gmax_of(as_v)

    def edge(i, _):
        s = srcv[pl.ds(i * L, L)]
        d = dstv[pl.ds(i * L, L)]
        a1 = plsc.load_gather(as_v, [s])
        a2 = plsc.load_gather(ad_v, [d])
        e = _leaky(a1 + a2)
        m = _leaky(gmax + a2)
        ex = jnp.exp(e - m)
        exv[pl.ds(i * L, L)] = ex
        plsc.addupdate_scatter(dloc, [d], ex)
        return _
    lax.fori_loop(0, EPT // L, edge, None)
    pltpu.sync_copy(exv, ex_hbm.at[pl.ds(wid * EPT, EPT)])

    pltpu.sync_copy(dloc, dsh.at[sid])
    plsc.subcore_barrier()

    def macc(t, _):
        pltpu.sync_copy(dsh.at[t, pl.ds(sid * SL, SL)], tmpv)

        def add16(i, __):
            accv[pl.ds(i * L, L)] = accv[pl.ds(i * L, L)] + tmpv[pl.ds(i * L, L)]
            return __
        lax.fori_loop(0, SL // L, add16, None)
        return _
    pltpu.sync_copy(dsh.at[0, pl.ds(sid * SL, SL)], accv)
    lax.fori_loop(1, NS, macc, None)
    pltpu.sync_copy(accv, den_hbm.at[cid, pl.ds(sid * SL, SL)])


_SC_PARAMS = pltpu.CompilerParams(needs_layout_passes=False)

_phase_a = functools.partial(
    pl.kernel,
    out_type=(jax.ShapeDtypeStruct((NC, NP), jnp.float32),
              jax.ShapeDtypeStruct((E2P,), jnp.float32)),
    mesh=plsc.VectorSubcoreMesh(core_axis_name="c", subcore_axis_name="s"),
    compiler_params=_SC_PARAMS,
    scratch_types=[
        pltpu.VMEM((NP,), jnp.float32),
        pltpu.VMEM((NP,), jnp.float32),
        pltpu.VMEM((EPT,), jnp.int32),
        pltpu.VMEM((EPT,), jnp.int32),
        pltpu.VMEM((EPT,), jnp.float32),
        pltpu.VMEM((NP,), jnp.float32),
        pltpu.VMEM((SL,), jnp.float32),
        pltpu.VMEM((SL,), jnp.float32),
        pltpu.VMEM_SHARED((NS, NP), jnp.float32),
        pltpu.SemaphoreType.DMA,
    ],
)(_phase_a_body)


def _phase_b_body(h_hbm, src_hbm, dst_hbm, ex_hbm, d2_hbm,
                  out_hbm,
                  den_v, tmpv, srcc, dstc, exc, rows,
                  acc_sh, sem):
    cid = lax.axis_index("c")
    sid = lax.axis_index("s")
    wid = cid * NS + sid
    pltpu.sync_copy(d2_hbm.at[0], den_v)
    pltpu.sync_copy(d2_hbm.at[1], tmpv)

    def addden(i, _):
        den_v[pl.ds(i * L, L)] = den_v[pl.ds(i * L, L)] + tmpv[pl.ds(i * L, L)]
        return _
    lax.fori_loop(0, NP // L, addden, None)

    def zrow(r, _):
        for c in range(D // L):
            plsc.store_scatter(
                rows, [jnp.full((L,), r, jnp.int32),
                       c * L + lax.iota(jnp.int32, L)],
                jnp.zeros((L,), jnp.float32))
        return _
    lax.fori_loop(0, K, zrow, None)
    for z in range(SL // K):
        pltpu.sync_copy(rows, acc_sh.at[pl.ds(sid * SL + z * K, K)])
    plsc.subcore_barrier()

    def chunk(j, _):
        base = wid * EPT + j * K
        pltpu.sync_copy(src_hbm.at[pl.ds(base, K)], srcc)
        pltpu.sync_copy(dst_hbm.at[pl.ds(base, K)], dstc)
        pltpu.sync_copy(ex_hbm.at[pl.ds(base, K)], exc)
        cp = pltpu.async_copy(h_hbm.at[srcc], rows, sem)
        for u in range(K // L):
            d = dstc[pl.ds(u * L, L)]
            dn = plsc.load_gather(den_v, [d])
            exc[pl.ds(u * L, L)] = exc[pl.ds(u * L, L)] / (dn + 1e-16)
        cp.wait()

        def scale(r, __):
            wb = plsc.load_gather(exc, [jnp.full((L,), r, jnp.int32)])
            ridx = jnp.full((L,), r, jnp.int32)
            for c in range(D // L):
                cidx = c * L + lax.iota(jnp.int32, L)
                v = plsc.load_gather(rows, [ridx, cidx])
                plsc.store_scatter(rows, [ridx, cidx], v * wb)
            return __
        lax.fori_loop(0, K, scale, None)
        pltpu.sync_copy(rows, acc_sh.at[dstc], add=True)
        return _
    lax.fori_loop(0, NCHUNK, chunk, None)

    plsc.subcore_barrier()
    pltpu.sync_copy(acc_sh.at[pl.ds(sid * SL, SL)],
                    out_hbm.at[cid, pl.ds(sid * SL, SL)])


_phase_b = functools.partial(
    pl.kernel,
    out_type=jax.ShapeDtypeStruct((NC, NP, D), jnp.float32),
    mesh=plsc.VectorSubcoreMesh(core_axis_name="c", subcore_axis_name="s"),
    compiler_params=_SC_PARAMS,
    scratch_types=[
        pltpu.VMEM((NP,), jnp.float32),
        pltpu.VMEM((NP,), jnp.float32),
        pltpu.VMEM((K,), jnp.int32),
        pltpu.VMEM((K,), jnp.int32),
        pltpu.VMEM((K,), jnp.float32),
        pltpu.VMEM((K, D), jnp.float32),
        pltpu.VMEM_SHARED((NP, D), jnp.float32),
        pltpu.SemaphoreType.DMA,
    ],
)(_phase_b_body)


def _dense1_body(x_ref, w_ref, asr_ref, adr_ref, h_ref, as_ref, ad_ref):
    h = jnp.dot(x_ref[...], w_ref[...], preferred_element_type=jnp.float32)
    h_ref[...] = h
    as_ref[...] = jnp.sum(h * asr_ref[...], axis=1, keepdims=True)
    ad_ref[...] = jnp.sum(h * adr_ref[...], axis=1, keepdims=True)


def _dense2_body(p_ref, b_ref, w_ref, asr_ref, adr_ref, h_ref, as_ref, ad_ref):
    i = pl.program_id(0)
    rows = i * _RB + lax.broadcasted_iota(jnp.int32, (_RB, D), 0)
    x = p_ref[0] + p_ref[1] + b_ref[...]
    x = jnp.where(rows < N, x, 0.0)
    h = jnp.dot(x, w_ref[...], preferred_element_type=jnp.float32)
    h_ref[...] = h
    as_ref[...] = jnp.sum(h * asr_ref[...], axis=1, keepdims=True)
    ad_ref[...] = jnp.sum(h * adr_ref[...], axis=1, keepdims=True)


_RB = 512


def _dense1(x, w, a_src, a_dst):
    return pl.pallas_call(
        _dense1_body,
        grid=(NP // _RB,),
        in_specs=[
            pl.BlockSpec((_RB, D), lambda i: (i, 0)),
            pl.BlockSpec((D, D), lambda i: (0, 0)),
            pl.BlockSpec((1, D), lambda i: (0, 0)),
            pl.BlockSpec((1, D), lambda i: (0, 0)),
        ],
        out_specs=[
            pl.BlockSpec((_RB, D), lambda i: (i, 0)),
            pl.BlockSpec((_RB, 1), lambda i: (i, 0)),
            pl.BlockSpec((_RB, 1), lambda i: (i, 0)),
        ],
        out_shape=[
            jax.ShapeDtypeStruct((NP, D), jnp.float32),
            jax.ShapeDtypeStruct((NP, 1), jnp.float32),
            jax.ShapeDtypeStruct((NP, 1), jnp.float32),
        ],
    )(x, w, a_src.reshape(1, D), a_dst.reshape(1, D))


def _dense2(p, b, w, a_src, a_dst):
    return pl.pallas_call(
        _dense2_body,
        grid=(NP // _RB,),
        in_specs=[
            pl.BlockSpec((NC, _RB, D), lambda i: (0, i, 0)),
            pl.BlockSpec((1, D), lambda i: (0, 0)),
            pl.BlockSpec((D, D), lambda i: (0, 0)),
            pl.BlockSpec((1, D), lambda i: (0, 0)),
            pl.BlockSpec((1, D), lambda i: (0, 0)),
        ],
        out_specs=[
            pl.BlockSpec((_RB, D), lambda i: (i, 0)),
            pl.BlockSpec((_RB, 1), lambda i: (i, 0)),
            pl.BlockSpec((_RB, 1), lambda i: (i, 0)),
        ],
        out_shape=[
            jax.ShapeDtypeStruct((NP, D), jnp.float32),
            jax.ShapeDtypeStruct((NP, 1), jnp.float32),
            jax.ShapeDtypeStruct((NP, 1), jnp.float32),
        ],
    )(p, b.reshape(1, D), w, a_src.reshape(1, D), a_dst.reshape(1, D))


_PB = 400


def _pool_body(p_ref, b_ref, batch_ref, out_ref, acc, cnt):
    i = pl.program_id(0)
    x = p_ref[0] + p_ref[1] + b_ref[...]
    onehot = (batch_ref[...] ==
              lax.broadcasted_iota(jnp.int32, (_PB, NG), 1)).astype(jnp.float32)
    psum = lax.dot_general(onehot, x, (((0,), (0,)), ((), ())),
                           preferred_element_type=jnp.float32)
    pcnt = lax.dot_general(onehot, jnp.ones((_PB, 1), jnp.float32),
                           (((0,), (0,)), ((), ())),
                           preferred_element_type=jnp.float32)

    @pl.when(i == 0)
    def _():
        acc[...] = jnp.zeros_like(acc)
        cnt[...] = jnp.zeros_like(cnt)

    acc[...] += psum
    cnt[...] += pcnt

    @pl.when(i == N // _PB - 1)
    def _():
        out_ref[...] = acc[...] / jnp.maximum(cnt[...], 1.0)


def _pool(p, b, batch):
    return pl.pallas_call(
        _pool_body,
        grid=(N // _PB,),
        in_specs=[
            pl.BlockSpec((NC, _PB, D), lambda i: (0, i, 0)),
            pl.BlockSpec((1, D), lambda i: (0, 0)),
            pl.BlockSpec((_PB, 1), lambda i: (i, 0)),
        ],
        out_specs=pl.BlockSpec((NG, D), lambda i: (0, 0)),
        out_shape=jax.ShapeDtypeStruct((NG, D), jnp.float32),
        scratch_shapes=[
            pltpu.VMEM((NG, D), jnp.float32),
            pltpu.VMEM((NG, 1), jnp.float32),
        ],
    )(p, b.reshape(1, D), batch.reshape(N, 1))


def kernel(x, edge_index, batch,
           W1, a_src1, a_dst1, b1, W2, a_src2, a_dst2, b2,
           W3, a_src3, a_dst3, b3):
    loop = jnp.arange(N, dtype=jnp.int32)
    padi = jnp.full((E2P - E - N,), N, jnp.int32)
    src = jnp.concatenate([edge_index[0], loop, padi])
    dst = jnp.concatenate([edge_index[1], loop, padi])
    xp = jnp.pad(x, ((0, NP - N), (0, 0)))

    h, asv, adv = _dense1(xp, W1, a_src1, a_dst1)
    for (w, a_s, a_d, b) in ((W2, a_src2, a_dst2, b1),
                             (W3, a_src3, a_dst3, b2)):
        den, ex = _phase_a(src, dst, asv.reshape(NP), adv.reshape(NP))
        p = _phase_b(h, src, dst, ex, den)
        h, asv, adv = _dense2(p, b, w, a_s, a_d)
    den, ex = _phase_a(src, dst, asv.reshape(NP), adv.reshape(NP))
    p = _phase_b(h, src, dst, ex, den)
    return _pool(p, b3, batch)
